# baseline (speedup 1.0000x reference)
import functools

import jax
import jax.numpy as jnp
from jax import lax
from jax.experimental import pallas as pl
from jax.experimental.pallas import tpu as pltpu
from jax.experimental.pallas import tpu_sc as plsc

SIDE = 28
N = SIDE * SIDE
CARD = 50
L = 16
NCHUNK = N // L
RMAIN = 48
RMAX = RMAIN + 8
W = 128

_INF = float("inf")


def _iota():
    return lax.iota(jnp.int32, L)


def _sort16(v):
    return jnp.sort(v)


def _permute(v, idx):
    return lax.gather(
        v, idx[:, None],
        dimension_numbers=lax.GatherDimensionNumbers(
            offset_dims=(), collapsed_slice_dims=(0,), start_index_map=(0,)),
        slice_sizes=(1,),
        mode=lax.GatherScatterMode.PROMISE_IN_BOUNDS)


def _sum_splat(v):
    it = _iota()
    for d in (8, 4, 2, 1):
        v = v + _permute(v, it ^ d)
    return v


def _merge32(s0, s1):
    rb = jnp.flip(s1)
    lo = jnp.minimum(s0, rb)
    hi = jnp.maximum(s0, rb)
    return _sort16(lo), _sort16(hi)


def _bitonic64_clean(l0, l1, l2, l3):
    m0 = jnp.minimum(l0, l2)
    m1 = jnp.minimum(l1, l3)
    M0 = jnp.maximum(l0, l2)
    M1 = jnp.maximum(l1, l3)
    u0 = jnp.minimum(m0, m1)
    u1 = jnp.maximum(m0, m1)
    v0 = jnp.minimum(M0, M1)
    v1 = jnp.maximum(M0, M1)
    return _sort16(u0), _sort16(u1), _sort16(v0), _sort16(v1)


def _merge64_keep_lo(a, b):
    lo0 = jnp.minimum(a[0], jnp.flip(b[3]))
    lo1 = jnp.minimum(a[1], jnp.flip(b[2]))
    lo2 = jnp.minimum(a[2], jnp.flip(b[1]))
    lo3 = jnp.minimum(a[3], jnp.flip(b[0]))
    return _bitonic64_clean(lo0, lo1, lo2, lo3)


def _sorted64_of_slice(x0, x1, x2, x3):
    s0, s1, s2, s3 = _sort16(x0), _sort16(x1), _sort16(x2), _sort16(x3)
    a = _merge32(s0, s1)
    b = _merge32(s2, s3)
    lo0 = jnp.minimum(a[0], jnp.flip(b[1]))
    lo1 = jnp.minimum(a[1], jnp.flip(b[0]))
    hi0 = jnp.maximum(a[0], jnp.flip(b[1]))
    hi1 = jnp.maximum(a[1], jnp.flip(b[0]))
    u0 = jnp.minimum(lo0, lo1)
    u1 = jnp.maximum(lo0, lo1)
    v0 = jnp.minimum(hi0, hi1)
    v1 = jnp.maximum(hi0, hi1)
    return _sort16(u0), _sort16(u1), _sort16(v0), _sort16(v1)


def _interleave_out(fin_v, res_v, out_hbm):
    it = _iota()
    zi = jnp.zeros((L,), jnp.int32)
    for k in range(4):
        rank = k * L + it
        mask = rank < CARD
        av = fin_v[0, pl.ds(k * L, L)]
        dv = fin_v[1, pl.ds(k * L, L)]
        plsc.store_scatter(res_v, [rank, zi], av, mask=mask)
        plsc.store_scatter(res_v, [rank, zi + 1], -dv, mask=mask)
    pltpu.sync_copy(res_v, out_hbm)


def _sc_body(p_hbm, I_hbm, J_hbm, out1_hbm, out2_hbm,
             p_v, mat_v, x64_v, xs_v, asc_v, dsc_v, ml_v,
             fin_v, res1_v, res2_v, xy_sh, sorted_sh, t64_sh):
    c = lax.axis_index("c")
    s = lax.axis_index("s")
    it = _iota()
    lane0 = it == 0

    pltpu.sync_copy(p_hbm, p_v)
    row_base = s * RMAIN
    tail = s >= 14

    pltpu.sync_copy(I_hbm.at[pl.ds(row_base, RMAIN)],
                    mat_v.at[pl.ds(0, RMAIN)])

    @pl.when(c != 0)
    def _():
        pltpu.sync_copy(J_hbm.at[pl.ds(row_base, RMAIN)],
                        mat_v.at[pl.ds(0, RMAIN)])

    tail_base = 8 * s + 656

    @pl.when(tail)
    def _():
        pltpu.sync_copy(I_hbm.at[pl.ds(tail_base, 8)],
                        mat_v.at[pl.ds(RMAIN, 8)])

        @pl.when(c != 0)
        def _():
            pltpu.sync_copy(J_hbm.at[pl.ds(tail_base, 8)],
                            mat_v.at[pl.ds(RMAIN, 8)])

    nrows = jnp.where(tail, RMAX, RMAIN)

    def pair_body(i, tot):
        r0 = 2 * i
        acc0 = jnp.zeros((L,), jnp.float32)
        acc1 = jnp.zeros((L,), jnp.float32)
        for ci in range(NCHUNK):
            pch = p_v[pl.ds(ci * L, L)]
            acc0 = acc0 + mat_v[r0, pl.ds(ci * L, L)] * pch
            acc1 = acc1 + mat_v[r0 + 1, pl.ds(ci * L, L)] * pch
        plsc.store_scatter(x64_v, [jnp.full((L,), r0, jnp.int32)],
                           _sum_splat(acc0), mask=lane0)
        plsc.store_scatter(x64_v, [jnp.full((L,), r0 + 1, jnp.int32)],
                           _sum_splat(acc1), mask=lane0)
        return tot

    lax.fori_loop(0, nrows // 2, pair_body, 0)
    pltpu.sync_copy(x64_v, xy_sh.at[s, pl.ds(0, 64)])

    plsc.subcore_barrier()

    pltpu.sync_copy(xy_sh.at[s, pl.ds(0, 64)], xs_v)
    inf_v = jnp.full((L,), _INF, jnp.float32)
    nreal3 = jnp.where(tail, 8, 0)
    m3 = it < nreal3

    x0 = xs_v[pl.ds(0, L)]
    x1 = xs_v[pl.ds(L, L)]
    x2 = xs_v[pl.ds(2 * L, L)]
    x3 = xs_v[pl.ds(3 * L, L)]

    a_list = _sorted64_of_slice(x0, x1, x2, jnp.where(m3, x3, inf_v))
    for k in range(4):
        asc_v[pl.ds(k * L, L)] = a_list[k]
    pltpu.sync_copy(asc_v, sorted_sh.at[0, s])

    d_list = _sorted64_of_slice(-x0, -x1, -x2, jnp.where(m3, -x3, inf_v))
    for k in range(4):
        dsc_v[pl.ds(k * L, L)] = d_list[k]
    pltpu.sync_copy(dsc_v, sorted_sh.at[1, s])

    plsc.subcore_barrier()

    @pl.when(s < 2)
    def _():
        pltpu.sync_copy(sorted_sh.at[s], ml_v)
        a = tuple(ml_v[0, pl.ds(k * L, L)] for k in range(4))

        def fold(g, a):
            b = tuple(ml_v[g, pl.ds(k * L, L)] for k in range(4))
            return _merge64_keep_lo(a, b)

        a = lax.fori_loop(1, L, fold, a)
        for k in range(4):
            asc_v[pl.ds(k * L, L)] = a[k]
        pltpu.sync_copy(asc_v, t64_sh.at[s])

    plsc.subcore_barrier()

    @pl.when((s == 0) & (c == 0))
    def _():
        pltpu.sync_copy(t64_sh, fin_v)
        _interleave_out(fin_v, res1_v, out1_hbm)

    @pl.when((s == 1) & (c != 0))
    def _():
        pltpu.sync_copy(t64_sh, fin_v)
        _interleave_out(fin_v, res2_v, out2_hbm)


@jax.jit
def _sc_call(p, I, J):
    f32 = jnp.float32
    out_sd = jax.ShapeDtypeStruct((CARD, 2), f32)
    fn = functools.partial(
        pl.kernel,
        out_type=(out_sd, out_sd),
        mesh=plsc.VectorSubcoreMesh(core_axis_name="c", subcore_axis_name="s"),
        compiler_params=pltpu.CompilerParams(needs_layout_passes=False),
        scratch_types=[
            pltpu.VMEM((N,), f32),
            pltpu.VMEM((RMAX, N), f32),
            pltpu.VMEM((64,), f32),
            pltpu.VMEM((64,), f32),
            pltpu.VMEM((W,), f32),
            pltpu.VMEM((W,), f32),
            pltpu.VMEM((L, W), f32),
            pltpu.VMEM((2, W), f32),
            pltpu.VMEM((CARD, 2), f32),
            pltpu.VMEM((CARD, 2), f32),
            pltpu.VMEM_SHARED((L, W), f32),
            pltpu.VMEM_SHARED((2, L, W), f32),
            pltpu.VMEM_SHARED((2, W), f32),
        ],
    )(_sc_body)
    return fn(p, I, J)


def kernel(p, I, J):
    return _sc_call(p, I, J)


# speedup vs baseline: 1.0282x; 1.0282x over previous
import jax
import jax.numpy as jnp
from jax.experimental import pallas as pl
from jax.experimental.pallas import tpu as pltpu

SIDE = 28
N = SIDE * SIDE
NPAD = 1024
CARD = 50
CHUNK = 128


def _tc_body(p_ref, I_ref, J_ref, dgm1_ref, dgm2_ref):
    p = p_ref[...]

    m = jax.lax.broadcasted_iota(jnp.int32, (128, 1), 0)
    k = m // 2
    tgt = jnp.where(m % 2 == 0, k, (N - 1) - k).astype(jnp.float32)

    ii = jax.lax.broadcasted_iota(jnp.int32, (N, CHUNK), 0)
    jj0 = jax.lax.broadcasted_iota(jnp.int32, (N, CHUNK), 1)
    jlt = [jj0 + (c * CHUNK) < ii for c in range(NPAD // CHUNK)]

    one_b = jnp.ones((N, CHUNK), jnp.bfloat16)
    zero_b = jnp.zeros((N, CHUNK), jnp.bfloat16)

    for mat_ref, out_ref in ((I_ref, dgm1_ref), (J_ref, dgm2_ref)):
        x = jax.lax.dot_general(
            mat_ref[...], p,
            dimension_numbers=(((1,), (0,)), ((), ())),
            preferred_element_type=jnp.float32,
        )
        xb = x.astype(jnp.bfloat16)
        xrow = xb.reshape(1, N)
        xrow = jnp.concatenate(
            [xrow, jnp.full((1, NPAD - N), jnp.inf, jnp.bfloat16)], axis=1)
        rank = jnp.zeros((N, 1), jnp.float32)
        for c in range(NPAD // CHUNK):
            xj = jax.lax.slice(xrow, (0, c * CHUNK), (1, (c + 1) * CHUNK))
            lt = (xj < xb)
            eq_lo = (xj == xb) & jlt[c]
            cnt = jnp.where(lt | eq_lo, one_b, zero_b)
            part = jnp.sum(cnt, axis=1, keepdims=True)
            rank = rank + part.astype(jnp.float32)
        onehot = (rank.reshape(1, N) == tgt).astype(jnp.float32)
        vals = jax.lax.dot_general(
            onehot, x,
            dimension_numbers=(((1,), (0,)), ((), ())),
            preferred_element_type=jnp.float32,
        )
        out_ref[...] = vals[: 2 * CARD].reshape(CARD, 2)


def kernel(p, I, J):
    p2 = p.reshape(N, 1)
    out_sd = jax.ShapeDtypeStruct((CARD, 2), jnp.float32)
    dgm1, dgm2 = pl.pallas_call(
        _tc_body,
        out_shape=(out_sd, out_sd),
    )(p2, I, J)
    return (dgm1, dgm2)


# speedup vs baseline: 2.5760x; 2.5053x over previous
import jax
import jax.numpy as jnp
from jax.experimental import pallas as pl
from jax.experimental.pallas import tpu as pltpu

SIDE = 28
N = SIDE * SIDE
NPAD = 1024
CARD = 50
CHUNK = 128


def _tc_body(p_ref, I_ref, J_ref, dgm1_ref, dgm2_ref):
    p = p_ref[...]

    m = jax.lax.broadcasted_iota(jnp.int32, (1, 128), 1)
    k = m // 2
    tgt = jnp.where(m % 2 == 0, k, (N - 1) - k).astype(jnp.float32)

    ii = jax.lax.broadcasted_iota(jnp.int32, (N, CHUNK), 0)
    jj0 = jax.lax.broadcasted_iota(jnp.int32, (N, CHUNK), 1)
    jlt = [jj0 + (c * CHUNK) < ii for c in range(NPAD // CHUNK)]

    for mat_ref, out_ref in ((I_ref, dgm1_ref), (J_ref, dgm2_ref)):
        x = jax.lax.dot_general(
            mat_ref[...], p,
            dimension_numbers=(((1,), (0,)), ((), ())),
            preferred_element_type=jnp.float32,
        )
        xrow = x.reshape(1, N)
        xrow = jnp.concatenate(
            [xrow, jnp.full((1, NPAD - N), jnp.inf, jnp.float32)], axis=1)
        rank = jnp.zeros((N, 1), jnp.float32)
        for c in range(NPAD // CHUNK):
            xj = jax.lax.slice(xrow, (0, c * CHUNK), (1, (c + 1) * CHUNK))
            lt = (xj < x)
            eq_lo = (xj == x) & jlt[c]
            cnt = (lt | eq_lo).astype(jnp.float32)
            rank = rank + jnp.sum(cnt, axis=1, keepdims=True)
        onehot = (rank == tgt).astype(jnp.float32)
        vals = jax.lax.dot_general(
            onehot, x,
            dimension_numbers=(((0,), (0,)), ((), ())),
            preferred_element_type=jnp.float32,
        )
        out_ref[...] = vals[: 2 * CARD].reshape(CARD, 2)


def kernel(p, I, J):
    p2 = p.reshape(N, 1)
    out_sd = jax.ShapeDtypeStruct((CARD, 2), jnp.float32)
    dgm1, dgm2 = pl.pallas_call(
        _tc_body,
        out_shape=(out_sd, out_sd),
    )(p2, I, J)
    return (dgm1, dgm2)


# speedup vs baseline: 3.0159x; 1.1708x over previous
import jax
import jax.numpy as jnp
from jax.experimental import pallas as pl
from jax.experimental.pallas import tpu as pltpu

SIDE = 28
N = SIDE * SIDE
NPAD = 1024
CARD = 50
CHUNK = 128


def _tc_body(p_ref, I_ref, J_ref, dgm1_ref, dgm2_ref):
    p = p_ref[...]

    m = jax.lax.broadcasted_iota(jnp.int32, (1, 128), 1)
    k = m // 2
    tgt = jnp.where(m % 2 == 0, k, (N - 1) - k).astype(jnp.float32)

    ii = jax.lax.broadcasted_iota(jnp.int32, (N, CHUNK), 0)
    jj0 = jax.lax.broadcasted_iota(jnp.int32, (N, CHUNK), 1)
    jlt = [jj0 + (c * CHUNK) < ii for c in range(NPAD // CHUNK)]

    for mat_ref, out_ref in ((I_ref, dgm1_ref), (J_ref, dgm2_ref)):
        x = jax.lax.dot_general(
            mat_ref[...], p,
            dimension_numbers=(((1,), (0,)), ((), ())),
            preferred_element_type=jnp.float32,
        )
        xrow = x.reshape(1, N)
        xrow = jnp.concatenate(
            [xrow, jnp.full((1, NPAD - N), jnp.inf, jnp.float32)], axis=1)
        acc = jnp.zeros((N, CHUNK), jnp.float32)
        for c in range(NPAD // CHUNK):
            xj = jax.lax.slice(xrow, (0, c * CHUNK), (1, (c + 1) * CHUNK))
            lt = (xj < x)
            eq_lo = (xj == x) & jlt[c]
            acc = acc + (lt | eq_lo).astype(jnp.float32)
        rank = jax.lax.dot_general(
            acc, jnp.ones((CHUNK, 1), jnp.float32),
            dimension_numbers=(((1,), (0,)), ((), ())),
            preferred_element_type=jnp.float32,
        )
        onehot = (rank == tgt).astype(jnp.float32)
        vals = jax.lax.dot_general(
            onehot, x,
            dimension_numbers=(((0,), (0,)), ((), ())),
            preferred_element_type=jnp.float32,
        )
        out_ref[...] = vals[: 2 * CARD].reshape(CARD, 2)


def kernel(p, I, J):
    p2 = p.reshape(N, 1)
    out_sd = jax.ShapeDtypeStruct((CARD, 2), jnp.float32)
    dgm1, dgm2 = pl.pallas_call(
        _tc_body,
        out_shape=(out_sd, out_sd),
    )(p2, I, J)
    return (dgm1, dgm2)
